# baseline (speedup 1.0000x reference)
import functools

import jax
import jax.numpy as jnp
from jax.experimental import pallas as pl
from jax.experimental.pallas import tpu as pltpu

_NUM_NOUN = 42
_NUM_VERB = 4
_CH = 256
_LATENT = 256
_NV1, _NV2, _NV3 = 1723, 431, 108
_EPS = 1e-5

_F32 = jnp.float32
_HI = jax.lax.Precision.HIGHEST


def _row(a):
    return a.reshape(1, -1)


def _dot(a, b, precision=None):
    return jnp.dot(a, b, precision=precision, preferred_element_type=_F32)


def _gn(y, g, b):
    V, C = y.shape
    s = jnp.sum(y, axis=0, keepdims=True)
    q = jnp.sum(y * y, axis=0, keepdims=True)
    gi = jax.lax.broadcasted_iota(jnp.int32, (C, C), 0) // 8
    gj = jax.lax.broadcasted_iota(jnp.int32, (C, C), 1) // 8
    P = jnp.where(gi == gj, 1.0, 0.0).astype(_F32)
    inv = 1.0 / (8.0 * V)
    m = _dot(s, P, precision=_HI) * inv
    e2 = _dot(q, P, precision=_HI) * inv
    var = e2 - m * m
    return (y - m) * jax.lax.rsqrt(var + _EPS) * g + b


def _grb(x, A, w, pre):
    g = lambda n: w[pre + n]
    y = jnp.maximum(_gn(x, g("preg"), g("preb")), 0.0)
    y = _dot(y, g("w1")) + g("b1")
    y = jnp.maximum(_gn(y, g("n1g"), g("n1b")), 0.0)
    y = _dot(A, _dot(y, g("wc"))) + g("bc")
    y = jnp.maximum(_gn(y, g("n2g"), g("n2b")), 0.0)
    y = _dot(y, g("w2")) + g("b2")
    if (pre + "ws") in w:
        x = _dot(x, g("ws")) + g("bs")
    return x + y


def _grb_flat(dst, pre, p):
    dst[pre + "preg"] = _row(p["pre_norm"]["g"])
    dst[pre + "preb"] = _row(p["pre_norm"]["b"])
    dst[pre + "w1"] = p["lin1"]["W"].T
    dst[pre + "b1"] = _row(p["lin1"]["b"])
    dst[pre + "n1g"] = _row(p["norm1"]["g"])
    dst[pre + "n1b"] = _row(p["norm1"]["b"])
    dst[pre + "wc"] = p["conv"]["W"].T
    dst[pre + "bc"] = _row(p["conv"]["b"])
    dst[pre + "n2g"] = _row(p["norm2"]["g"])
    dst[pre + "n2b"] = _row(p["norm2"]["b"])
    dst[pre + "w2"] = p["lin2"]["W"].T
    dst[pre + "b2"] = _row(p["lin2"]["b"])
    if "skip" in p:
        dst[pre + "ws"] = p["skip"]["W"].T
        dst[pre + "bs"] = _row(p["skip"]["b"])


def _const_map(ndim):
    return lambda b: (0,) * ndim


def _enc_body(names, xc_ref, ic_ref, *rest):
    wrefs, out_ref = rest[:-1], rest[-1]
    w = {n: r[...] for n, r in zip(names, wrefs)}
    xc = xc_ref[0]
    ic = ic_ref[0]
    y = _dot(xc, w["Wxc"]) + _dot(ic, w["Wic"]) + w["b0"]
    y = _grb(y, w["A1"], w, "e0_")
    y = _grb(y, w["A1"], w, "e1_")
    y = _dot(w["D1"], y)
    y = _grb(y, w["A2"], w, "e2_")
    y = _grb(y, w["A2"], w, "e3_")
    y = _dot(w["D2"], y)
    y = _grb(y, w["A3"], w, "e4_")
    y = _grb(y, w["A3"], w, "e5_")
    out_ref[0] = y


def _enc_call(xc, ic3, w):
    names = tuple(sorted(w))
    ws = [w[n] for n in names]
    Bn = xc.shape[0]
    in_specs = [
        pl.BlockSpec((1,) + xc.shape[1:], lambda b: (b, 0, 0)),
        pl.BlockSpec((1,) + ic3.shape[1:], lambda b: (b, 0, 0)),
    ] + [pl.BlockSpec(a.shape, _const_map(a.ndim)) for a in ws]
    return pl.pallas_call(
        functools.partial(_enc_body, names),
        grid=(Bn,),
        in_specs=in_specs,
        out_specs=pl.BlockSpec((1, _NV3, _CH), lambda b: (b, 0, 0)),
        out_shape=jax.ShapeDtypeStruct((Bn, _NV3, _CH), _F32),
        compiler_params=pltpu.CompilerParams(
            dimension_semantics=("arbitrary",),
            vmem_limit_bytes=100 * 1024 * 1024,
        ),
    )(xc, ic3, *ws)


def _lat_body(f_ref, w_ref, b_ref, o_ref):
    o_ref[...] = _dot(f_ref[...], w_ref[...]) + b_ref[...]


def _lat_call(feat, Wl, bl):
    return pl.pallas_call(
        _lat_body,
        out_shape=jax.ShapeDtypeStruct((feat.shape[0], _LATENT), _F32),
        compiler_params=pltpu.CompilerParams(
            vmem_limit_bytes=100 * 1024 * 1024,
        ),
    )(feat, Wl, bl)


def _dec_body(names, z_ref, ic_ref, *rest):
    wrefs, ox_ref, of_ref = rest[:-2], rest[-2], rest[-1]
    w = {n: r[...] for n, r in zip(names, wrefs)}
    z = z_ref[0]
    ic = ic_ref[0]
    y = _dot(z, w["Wz"]) + _dot(ic, w["Wicd"]) + _dot(w["refT"], w["Wrefd"]) + w["bd"]
    y = _grb(y, w["A3"], w, "d0_")
    y = _grb(y, w["A3"], w, "d1_")
    y = _dot(w["U2"], y)
    y = _grb(y, w["A2"], w, "d2_")
    y = _grb(y, w["A2"], w, "d3_")
    y = _dot(w["U1"], y)
    y = _grb(y, w["A1"], w, "d4_")
    y = _grb(y, w["A1"], w, "d5_")
    y = _grb(y, w["A1"], w, "ga_")
    y = _grb(y, w["A1"], w, "gb_")
    y = jnp.maximum(_gn(y, w["fing"], w["finb"]), 0.0)
    px = _dot(y, w["Wo1"]) + w["bo1"]
    pf = _dot(y, w["Wo2"]) + w["bo2"]
    r = _dot(ic, w["Wicr"]) + _dot(w["rinitT"], w["Wrr"]) + w["br0"]
    r = jnp.maximum(r * w["bn0g"] + w["bn0b"], 0.0)
    for blk in ("r0", "r1"):
        t = _dot(r, w[blk + "c1w"]) + w[blk + "c1b"]
        t = jnp.maximum(t * w[blk + "bn1g"] + w[blk + "bn1b"], 0.0)
        t = _dot(t, w[blk + "c2w"]) + w[blk + "c2b"]
        t = t * w[blk + "bn2g"] + w[blk + "bn2b"]
        r = jnp.maximum(r + t, 0.0)
    r = _dot(r, w["Wor"]) + w["bor"]
    pf = pf + r
    lane = jax.lax.broadcasted_iota(jnp.int32, pf.shape, 1)
    f = jnp.where(lane == 0, 1.0 / (1.0 + jnp.exp(-pf)), pf)
    ox_ref[0] = px
    of_ref[0] = f


def _dec_call(z3, ic3, w):
    names = tuple(sorted(w))
    ws = [w[n] for n in names]
    Bn = z3.shape[0]
    in_specs = [
        pl.BlockSpec((1,) + z3.shape[1:], lambda b: (b, 0, 0)),
        pl.BlockSpec((1,) + ic3.shape[1:], lambda b: (b, 0, 0)),
    ] + [pl.BlockSpec(a.shape, _const_map(a.ndim)) for a in ws]
    return pl.pallas_call(
        functools.partial(_dec_body, names),
        grid=(Bn,),
        in_specs=in_specs,
        out_specs=[
            pl.BlockSpec((1, _NV1, 3), lambda b: (b, 0, 0)),
            pl.BlockSpec((1, _NV1, _NUM_NOUN + 1), lambda b: (b, 0, 0)),
        ],
        out_shape=[
            jax.ShapeDtypeStruct((Bn, _NV1, 3), _F32),
            jax.ShapeDtypeStruct((Bn, _NV1, _NUM_NOUN + 1), _F32),
        ],
        compiler_params=pltpu.CompilerParams(
            dimension_semantics=("arbitrary",),
            vmem_limit_bytes=100 * 1024 * 1024,
        ),
    )(z3, ic3, *ws)


def kernel(body_vertices, contact_features, interaction_code, params, bufs):
    Bn = body_vertices.shape[0]
    bnscale = 1.0 / jnp.sqrt(jnp.float32(1.0 + _EPS))

    xc = jnp.concatenate([body_vertices, contact_features], axis=2)
    ic3 = interaction_code[:, None, :]

    ew = {
        "A1": bufs["A1"], "A2": bufs["A2"], "A3": bufs["A3"],
        "D1": bufs["D1"], "D2": bufs["D2"],
        "Wxc": params["enc_gl"]["W"][:, :46].T,
        "Wic": params["enc_gl"]["W"][:, 46:].T,
        "b0": _row(params["enc_gl"]["b"]),
    }
    for i, p in enumerate(params["enc_grb"]):
        _grb_flat(ew, f"e{i}_", p)

    y3 = _enc_call(xc, ic3, ew)

    Wl = params["latent"]["W"].reshape(_LATENT, _CH, _NV3)
    Wl = Wl.transpose(2, 1, 0).reshape(_NV3 * _CH, _LATENT)
    z = _lat_call(y3.reshape(Bn, _NV3 * _CH), Wl, _row(params["latent"]["b"]))

    Wd = params["dec_gl"]["W"]
    dw = {
        "A1": bufs["A1"], "A2": bufs["A2"], "A3": bufs["A3"],
        "U1": bufs["U1"], "U2": bufs["U2"],
        "refT": bufs["ref"].T, "rinitT": bufs["ref_init"].T,
        "Wz": Wd[:, :_LATENT].T,
        "Wicd": Wd[:, _LATENT:_LATENT + _NUM_VERB * _NUM_NOUN].T,
        "Wrefd": Wd[:, _LATENT + _NUM_VERB * _NUM_NOUN:].T,
        "bd": _row(params["dec_gl"]["b"]),
        "fing": _row(params["gn_final"]["g"]),
        "finb": _row(params["gn_final"]["b"]),
        "Wo1": params["out_gl"]["W"][:3].T,
        "bo1": _row(params["out_gl"]["b"][:3]),
        "Wo2": params["out_gl"]["W"][3:].T,
        "bo2": _row(params["out_gl"]["b"][3:]),
    }
    for i, p in enumerate(params["dec_grb"]):
        _grb_flat(dw, f"d{i}_", p)
    _grb_flat(dw, "ga_", params["grb_a"])
    _grb_flat(dw, "gb_", params["grb_b"])

    res = params["res"]
    Wf = res["fc0"]["W"]
    dw.update({
        "Wicr": Wf[:, :_NUM_VERB * _NUM_NOUN].T,
        "Wrr": Wf[:, _NUM_VERB * _NUM_NOUN:].T,
        "br0": _row(res["fc0"]["b"]),
        "bn0g": _row(res["bn0"]["g"]) * bnscale,
        "bn0b": _row(res["bn0"]["b"]),
        "Wor": res["out"]["W"].T,
        "bor": _row(res["out"]["b"]),
    })
    for j, blk in enumerate(res["blocks"]):
        pre = f"r{j}"
        dw.update({
            pre + "c1w": blk["c1"]["W"].T, pre + "c1b": _row(blk["c1"]["b"]),
            pre + "bn1g": _row(blk["bn1"]["g"]) * bnscale,
            pre + "bn1b": _row(blk["bn1"]["b"]),
            pre + "c2w": blk["c2"]["W"].T, pre + "c2b": _row(blk["c2"]["b"]),
            pre + "bn2g": _row(blk["bn2"]["g"]) * bnscale,
            pre + "bn2b": _row(blk["bn2"]["b"]),
        })

    x_rec, f = _dec_call(z[:, None, :], ic3, dw)
    return x_rec, f
